# baseline (speedup 1.0000x reference)
import jax
import jax.numpy as jnp
from jax.experimental import pallas as pl

H = 32
W = 32
D = 256


BB = 4


def _body(col_ref, row_ref, out_ref):
    col = col_ref[...]
    for bb in range(BB):
        for h in range(H):
            out_ref[bb, h, :, :D] = col
            out_ref[bb, h, :, D:] = jnp.broadcast_to(
                row_ref[h, :][None, :], (W, D)
            )


def kernel(x, row_embed, col_embed):
    b = x.shape[0]
    out = pl.pallas_call(
        _body,
        grid=(b // BB,),
        in_specs=[
            pl.BlockSpec((W, D), lambda i: (0, 0)),
            pl.BlockSpec((H, D), lambda i: (0, 0)),
        ],
        out_specs=pl.BlockSpec((BB, H, W, 2 * D), lambda i: (i, 0, 0, 0)),
        out_shape=jax.ShapeDtypeStruct((b, H, W, 2 * D), jnp.float32),
    )(col_embed, row_embed)
    return jnp.transpose(out, (0, 3, 1, 2))
